# baseline (speedup 1.0000x reference)
import functools

import jax
import jax.numpy as jnp
from jax import lax
from jax.experimental import pallas as pl
from jax.experimental.pallas import tpu as pltpu
from jax.experimental.pallas import tpu_sc as plsc

N = 10000
E = 320000
IN_DIM = 128
HID = 64
NC = 2
NS = 16
NW = NC * NS
CHUNK = 128
NCHUNK = 80
EPAD = NW * NCHUNK * CHUNK
NPAD = 10240
RPT = NPAD // NS

_mesh = plsc.VectorSubcoreMesh(core_axis_name="c", subcore_axis_name="s")
_sc_params = pltpu.CompilerParams(use_tc_tiling_on_sc=False)


DEGW = 16

@functools.partial(
    pl.kernel,
    out_type=jax.ShapeDtypeStruct((NC, NPAD, DEGW), jnp.float32),
    mesh=_mesh,
    scratch_types=[
        pltpu.VMEM((NCHUNK, CHUNK), jnp.int32),
        pltpu.VMEM((CHUNK, DEGW), jnp.float32),
        pltpu.VMEM_SHARED((NPAD, DEGW), jnp.float32),
        pltpu.SemaphoreType.DMA,
    ],
    compiler_params=_sc_params,
)
def _deg_kernel(dst_hbm, ones_hbm, z_hbm, out_hbm, dst_v, ones_v, deg_sp, sem):
    del sem
    cid = lax.axis_index("c")
    tid = lax.axis_index("s")
    wid = cid * NS + tid
    sl = pl.ds(tid * RPT, RPT)
    pltpu.sync_copy(z_hbm, deg_sp.at[sl])
    pltpu.sync_copy(dst_hbm.at[wid], dst_v)
    pltpu.sync_copy(ones_hbm, ones_v)
    plsc.subcore_barrier()

    def body(c, carry):
        pltpu.sync_copy(ones_v, deg_sp.at[dst_v.at[c]], add=True)
        return carry

    lax.fori_loop(0, NCHUNK, body, 0)
    plsc.subcore_barrier()
    pltpu.sync_copy(deg_sp.at[sl], out_hbm.at[cid, sl])


@functools.partial(
    pl.kernel,
    out_type=jax.ShapeDtypeStruct((NC, NPAD, HID), jnp.float32),
    mesh=_mesh,
    scratch_types=[
        pltpu.VMEM((NCHUNK, CHUNK), jnp.int32),
        pltpu.VMEM((NCHUNK, CHUNK), jnp.int32),
        pltpu.VMEM((2, CHUNK, HID), jnp.float32),
        pltpu.VMEM_SHARED((NPAD, HID), jnp.float32),
        pltpu.SemaphoreType.DMA,
    ],
    compiler_params=_sc_params,
)
def _edge_kernel(p_hbm, src_hbm, dst_hbm, z_hbm, out_hbm,
                 src_v, dst_v, rows2, acc_sp, sem):
    cid = lax.axis_index("c")
    tid = lax.axis_index("s")
    wid = cid * NS + tid
    sl = pl.ds(tid * RPT, RPT)
    pltpu.sync_copy(z_hbm, acc_sp.at[sl])
    pltpu.sync_copy(src_hbm.at[wid], src_v)
    pltpu.sync_copy(dst_hbm.at[wid], dst_v)
    plsc.subcore_barrier()

    pltpu.async_copy(p_hbm.at[src_v.at[0]], rows2.at[0], sem)

    def step(c):
        buf = c % 2
        pltpu.make_async_copy(p_hbm.at[src_v.at[c]], rows2.at[buf], sem).wait()
        pltpu.sync_copy(rows2.at[buf], acc_sp.at[dst_v.at[c]], add=True)

    def body(c, carry):
        pltpu.async_copy(p_hbm.at[src_v.at[c + 1]], rows2.at[(c + 1) % 2], sem)
        step(c)
        return carry

    lax.fori_loop(0, NCHUNK - 1, body, 0)
    step(NCHUNK - 1)
    plsc.subcore_barrier()
    pltpu.sync_copy(acc_sp.at[sl], out_hbm.at[cid, sl])


def _prep_body(x_ref, w_ref, deg_ref, p_ref, dinv_ref):
    deg = deg_ref[0, :, 0:1] + deg_ref[1, :, 0:1] + 1.0
    dinv = lax.rsqrt(jnp.maximum(deg, 1.0))
    dinv_ref[...] = dinv
    p_ref[...] = jnp.dot(x_ref[...], w_ref[...],
                         preferred_element_type=jnp.float32) * dinv


_prep = pl.pallas_call(
    _prep_body,
    out_shape=[
        jax.ShapeDtypeStruct((NPAD, HID), jnp.float32),
        jax.ShapeDtypeStruct((NPAD, 1), jnp.float32),
    ],
)


def _mid_body(acc_ref, p_ref, dinv_ref, b_ref, g_ref, be_ref, w_ref, out_ref):
    dinv = dinv_ref[...]
    z = dinv * (acc_ref[0] + acc_ref[1] + p_ref[...]) + b_ref[...]
    rows = lax.broadcasted_iota(jnp.int32, (NPAD, HID), 0)
    mask = rows < N
    zm = jnp.where(mask, z, 0.0)
    mu = jnp.sum(zm, axis=0, keepdims=True) * (1.0 / N)
    d = jnp.where(mask, z - mu, 0.0)
    var = jnp.sum(d * d, axis=0, keepdims=True) * (1.0 / N)
    y = g_ref[...] * (z - mu) * lax.rsqrt(var + 1e-5) + be_ref[...]
    y = jnp.maximum(y, 0.0)
    out_ref[...] = jnp.dot(y, w_ref[...],
                           preferred_element_type=jnp.float32) * dinv


_mid = pl.pallas_call(
    _mid_body,
    out_shape=jax.ShapeDtypeStruct((NPAD, HID), jnp.float32),
)


def _head_body(acc_ref, p_ref, dinv_ref, b_ref, ws_ref, bs_ref, wa_ref, ba_ref,
               s_ref, a_ref):
    z = dinv_ref[...] * (acc_ref[0] + acc_ref[1] + p_ref[...]) + b_ref[...]
    s_ref[...] = jnp.dot(z, ws_ref[...],
                         preferred_element_type=jnp.float32) + bs_ref[...]
    a_ref[...] = jnp.dot(z, wa_ref[...],
                         preferred_element_type=jnp.float32) + ba_ref[...]


_head = pl.pallas_call(
    _head_body,
    out_shape=[
        jax.ShapeDtypeStruct((NPAD, 4), jnp.float32),
        jax.ShapeDtypeStruct((NPAD, 4), jnp.float32),
    ],
)


def kernel(x, edge_index, W1, b1, W2, b2, W3, b3, gamma, beta, Ws, bs, Wa, ba):
    padw = (EPAD - E) // NW
    pad_src = jnp.zeros((NW, padw), jnp.int32)
    pad_dst = jnp.broadcast_to(N + jnp.arange(padw, dtype=jnp.int32), (NW, padw))
    src2 = jnp.concatenate([edge_index[0].reshape(NW, E // NW), pad_src],
                           axis=1).reshape(NW, NCHUNK, CHUNK)
    dst2 = jnp.concatenate([edge_index[1].reshape(NW, E // NW), pad_dst],
                           axis=1).reshape(NW, NCHUNK, CHUNK)
    x_pad = jnp.pad(x, ((0, NPAD - N), (0, 0)))
    ones = jnp.ones((CHUNK, DEGW), jnp.float32)
    z1 = jnp.zeros((RPT, DEGW), jnp.float32)
    z64 = jnp.zeros((RPT, HID), jnp.float32)
    b1r, b2r, b3r = b1.reshape(1, HID), b2.reshape(1, HID), b3.reshape(1, HID)
    gr, ber = gamma.reshape(1, HID), beta.reshape(1, HID)
    bsr, bar = bs.reshape(1, 4), ba.reshape(1, 4)

    deg2 = _deg_kernel(dst2, ones, z1)
    p1, dinv = _prep(x_pad, W1, deg2)

    acc1 = _edge_kernel(p1, src2, dst2, z64)
    p2 = _mid(acc1, p1, dinv, b1r, gr, ber, W2)
    acc2 = _edge_kernel(p2, src2, dst2, z64)
    p3 = _mid(acc2, p2, dinv, b2r, gr, ber, W3)
    acc3 = _edge_kernel(p3, src2, dst2, z64)
    sizes, aligns = _head(acc3, p3, dinv, b3r, Ws, bsr, Wa, bar)
    return (sizes[:N], aligns[:N])


# speedup vs baseline: 1.9280x; 1.9280x over previous
import functools

import jax
import jax.numpy as jnp
from jax import lax
from jax.experimental import pallas as pl
from jax.experimental.pallas import tpu as pltpu
from jax.experimental.pallas import tpu_sc as plsc

N = 10000
E = 320000
IN_DIM = 128
HID = 64
NC = 2
NS = 16
NW = NC * NS
CHUNK = 80
NCHUNK = 125
EPAD = NW * NCHUNK * CHUNK
NPAD = 10240
RPT = NPAD // NS

_mesh = plsc.VectorSubcoreMesh(core_axis_name="c", subcore_axis_name="s")
_sc_params = pltpu.CompilerParams(use_tc_tiling_on_sc=False)


DEGW = 16

@functools.partial(
    pl.kernel,
    out_type=jax.ShapeDtypeStruct((NC, NPAD, DEGW), jnp.float32),
    mesh=_mesh,
    scratch_types=[
        pltpu.VMEM((NCHUNK, CHUNK), jnp.int32),
        pltpu.VMEM((CHUNK, DEGW), jnp.float32),
        pltpu.VMEM_SHARED((NPAD, DEGW), jnp.float32),
        pltpu.SemaphoreType.DMA,
    ],
    compiler_params=_sc_params,
)
def _deg_kernel(dst_hbm, ones_hbm, z_hbm, out_hbm, dst_v, ones_v, deg_sp, sem):
    del sem
    cid = lax.axis_index("c")
    tid = lax.axis_index("s")
    wid = cid * NS + tid
    sl = pl.ds(tid * RPT, RPT)
    pltpu.sync_copy(z_hbm, deg_sp.at[sl])
    pltpu.sync_copy(dst_hbm.at[wid], dst_v)
    pltpu.sync_copy(ones_hbm, ones_v)
    plsc.subcore_barrier()

    def body(c, carry):
        pltpu.sync_copy(ones_v, deg_sp.at[dst_v.at[c]], add=True)
        return carry

    lax.fori_loop(0, NCHUNK, body, 0)
    plsc.subcore_barrier()
    pltpu.sync_copy(deg_sp.at[sl], out_hbm.at[cid, sl])


@functools.partial(
    pl.kernel,
    out_type=jax.ShapeDtypeStruct((NC, NPAD, HID), jnp.float32),
    mesh=_mesh,
    scratch_types=[
        pltpu.VMEM((NCHUNK, CHUNK), jnp.int32),
        pltpu.VMEM((NCHUNK, CHUNK), jnp.int32),
        pltpu.VMEM((2, CHUNK, HID), jnp.float32),
        pltpu.VMEM_SHARED((NPAD, HID), jnp.float32),
        pltpu.SemaphoreType.DMA,
    ],
    compiler_params=_sc_params,
)
def _edge_kernel(p_hbm, src_hbm, dst_hbm, z_hbm, out_hbm,
                 src_v, dst_v, rows2, acc_sp, sem):
    cid = lax.axis_index("c")
    tid = lax.axis_index("s")
    wid = cid * NS + tid
    sl = pl.ds(tid * RPT, RPT)
    pltpu.sync_copy(z_hbm, acc_sp.at[sl])
    pltpu.sync_copy(src_hbm.at[wid], src_v)
    pltpu.sync_copy(dst_hbm.at[wid], dst_v)
    plsc.subcore_barrier()

    pltpu.async_copy(p_hbm.at[src_v.at[0]], rows2.at[0], sem)

    def step(c):
        buf = c % 2
        pltpu.make_async_copy(p_hbm.at[src_v.at[c]], rows2.at[buf], sem).wait()
        pltpu.sync_copy(rows2.at[buf], acc_sp.at[dst_v.at[c]], add=True)

    def body(c, carry):
        pltpu.async_copy(p_hbm.at[src_v.at[c + 1]], rows2.at[(c + 1) % 2], sem)
        step(c)
        return carry

    lax.fori_loop(0, NCHUNK - 1, body, 0)
    step(NCHUNK - 1)
    plsc.subcore_barrier()
    pltpu.sync_copy(acc_sp.at[sl], out_hbm.at[cid, sl])


def _prep_body(x_ref, w_ref, deg_ref, p_ref, dinv_ref):
    deg = deg_ref[0, :, 0:1] + deg_ref[1, :, 0:1] + 1.0
    dinv = lax.rsqrt(jnp.maximum(deg, 1.0))
    dinv_ref[...] = dinv
    p_ref[...] = jnp.dot(x_ref[...], w_ref[...],
                         preferred_element_type=jnp.float32) * dinv


_prep = pl.pallas_call(
    _prep_body,
    out_shape=[
        jax.ShapeDtypeStruct((NPAD, HID), jnp.float32),
        jax.ShapeDtypeStruct((NPAD, 1), jnp.float32),
    ],
)


def _mid_body(acc_ref, p_ref, dinv_ref, b_ref, g_ref, be_ref, w_ref, out_ref):
    dinv = dinv_ref[...]
    z = dinv * (acc_ref[0] + acc_ref[1] + p_ref[...]) + b_ref[...]
    rows = lax.broadcasted_iota(jnp.int32, (NPAD, HID), 0)
    mask = rows < N
    zm = jnp.where(mask, z, 0.0)
    mu = jnp.sum(zm, axis=0, keepdims=True) * (1.0 / N)
    d = jnp.where(mask, z - mu, 0.0)
    var = jnp.sum(d * d, axis=0, keepdims=True) * (1.0 / N)
    y = g_ref[...] * (z - mu) * lax.rsqrt(var + 1e-5) + be_ref[...]
    y = jnp.maximum(y, 0.0)
    out_ref[...] = jnp.dot(y, w_ref[...],
                           preferred_element_type=jnp.float32) * dinv


_mid = pl.pallas_call(
    _mid_body,
    out_shape=jax.ShapeDtypeStruct((NPAD, HID), jnp.float32),
)


def _head_body(acc_ref, p_ref, dinv_ref, b_ref, ws_ref, bs_ref, wa_ref, ba_ref,
               s_ref, a_ref):
    z = dinv_ref[...] * (acc_ref[0] + acc_ref[1] + p_ref[...]) + b_ref[...]
    s_ref[...] = jnp.dot(z, ws_ref[...],
                         preferred_element_type=jnp.float32) + bs_ref[...]
    a_ref[...] = jnp.dot(z, wa_ref[...],
                         preferred_element_type=jnp.float32) + ba_ref[...]


_head = pl.pallas_call(
    _head_body,
    out_shape=[
        jax.ShapeDtypeStruct((NPAD, 4), jnp.float32),
        jax.ShapeDtypeStruct((NPAD, 4), jnp.float32),
    ],
)


def kernel(x, edge_index, W1, b1, W2, b2, W3, b3, gamma, beta, Ws, bs, Wa, ba):
    src2 = edge_index[0].reshape(NW, NCHUNK, CHUNK)
    dst2 = edge_index[1].reshape(NW, NCHUNK, CHUNK)
    x_pad = jnp.pad(x, ((0, NPAD - N), (0, 0)))
    ones = jnp.ones((CHUNK, DEGW), jnp.float32)
    z1 = jnp.zeros((RPT, DEGW), jnp.float32)
    z64 = jnp.zeros((RPT, HID), jnp.float32)
    b1r, b2r, b3r = b1.reshape(1, HID), b2.reshape(1, HID), b3.reshape(1, HID)
    gr, ber = gamma.reshape(1, HID), beta.reshape(1, HID)
    bsr, bar = bs.reshape(1, 4), ba.reshape(1, 4)

    deg2 = _deg_kernel(dst2, ones, z1)
    p1, dinv = _prep(x_pad, W1, deg2)

    acc1 = _edge_kernel(p1, src2, dst2, z64)
    p2 = _mid(acc1, p1, dinv, b1r, gr, ber, W2)
    acc2 = _edge_kernel(p2, src2, dst2, z64)
    p3 = _mid(acc2, p2, dinv, b2r, gr, ber, W3)
    acc3 = _edge_kernel(p3, src2, dst2, z64)
    sizes, aligns = _head(acc3, p3, dinv, b3r, Ws, bsr, Wa, bar)
    return (sizes[:N], aligns[:N])


# speedup vs baseline: 2.4068x; 1.2484x over previous
import functools

import jax
import jax.numpy as jnp
from jax import lax
from jax.experimental import pallas as pl
from jax.experimental.pallas import tpu as pltpu
from jax.experimental.pallas import tpu_sc as plsc

N = 10000
E = 320000
IN_DIM = 128
HID = 64
NC = 2
NS = 16
NW = NC * NS
CHUNK = 80
NCHUNK = 125
EPAD = NW * NCHUNK * CHUNK
NPAD = 10240
RPT = NPAD // NS

_mesh = plsc.VectorSubcoreMesh(core_axis_name="c", subcore_axis_name="s")
_sc_params = pltpu.CompilerParams(use_tc_tiling_on_sc=False)


DEGW = 16

@functools.partial(
    pl.kernel,
    out_type=jax.ShapeDtypeStruct((NC, NPAD, DEGW), jnp.float32),
    mesh=_mesh,
    scratch_types=[
        pltpu.VMEM((NCHUNK, CHUNK), jnp.int32),
        pltpu.VMEM((CHUNK, DEGW), jnp.float32),
        pltpu.VMEM_SHARED((NPAD, DEGW), jnp.float32),
        pltpu.SemaphoreType.DMA,
    ],
    compiler_params=_sc_params,
)
def _deg_kernel(dst_hbm, ones_hbm, z_hbm, out_hbm, dst_v, ones_v, deg_sp, sem):
    del sem
    cid = lax.axis_index("c")
    tid = lax.axis_index("s")
    wid = cid * NS + tid
    sl = pl.ds(tid * RPT, RPT)
    pltpu.sync_copy(z_hbm, deg_sp.at[sl])
    pltpu.sync_copy(dst_hbm.at[wid], dst_v)
    pltpu.sync_copy(ones_hbm, ones_v)
    plsc.subcore_barrier()

    def body(c, carry):
        pltpu.sync_copy(ones_v, deg_sp.at[dst_v.at[c]], add=True)
        return carry

    lax.fori_loop(0, NCHUNK, body, 0)
    plsc.subcore_barrier()
    pltpu.sync_copy(deg_sp.at[sl], out_hbm.at[cid, sl])


@functools.partial(
    pl.kernel,
    out_type=jax.ShapeDtypeStruct((NC, NPAD, HID), jnp.float32),
    mesh=_mesh,
    scratch_types=[
        pltpu.VMEM((NCHUNK, CHUNK), jnp.int32),
        pltpu.VMEM((NCHUNK, CHUNK), jnp.int32),
        pltpu.VMEM((4, CHUNK, HID), jnp.float32),
        pltpu.VMEM_SHARED((NPAD, HID), jnp.float32),
        pltpu.SemaphoreType.DMA,
        pltpu.SemaphoreType.DMA,
    ],
    compiler_params=_sc_params,
)
def _edge_kernel(p_hbm, src_hbm, dst_hbm, z_hbm, out_hbm,
                 src_v, dst_v, rows4, acc_sp, sem_g, sem_s):
    cid = lax.axis_index("c")
    tid = lax.axis_index("s")
    wid = cid * NS + tid
    sl = pl.ds(tid * RPT, RPT)
    pltpu.sync_copy(z_hbm, acc_sp.at[sl])
    pltpu.sync_copy(src_hbm.at[wid], src_v)
    pltpu.sync_copy(dst_hbm.at[wid], dst_v)
    plsc.subcore_barrier()

    def g_desc(c):
        return pltpu.make_async_copy(p_hbm.at[src_v.at[c]], rows4.at[c % 4],
                                     sem_g)

    def s_desc(c):
        return pltpu.make_async_copy(rows4.at[c % 4], acc_sp.at[dst_v.at[c]],
                                     sem_s)

    g_desc(0).start()
    g_desc(1).start()

    def body(c, carry):
        @pl.when(c >= 2)
        def _():
            s_desc(c - 2).wait()

        @pl.when(c + 2 < NCHUNK)
        def _():
            g_desc(c + 2).start()

        g_desc(c).wait()
        s_desc(c).start(add=True)
        return carry

    lax.fori_loop(0, NCHUNK, body, 0)
    s_desc(NCHUNK - 2).wait()
    s_desc(NCHUNK - 1).wait()
    plsc.subcore_barrier()
    pltpu.sync_copy(acc_sp.at[sl], out_hbm.at[cid, sl])


def _prep_body(x_ref, w_ref, deg_ref, p_ref, dinv_ref):
    deg = deg_ref[0, :, 0:1] + deg_ref[1, :, 0:1] + 1.0
    dinv = lax.rsqrt(jnp.maximum(deg, 1.0))
    dinv_ref[...] = dinv
    p_ref[...] = jnp.dot(x_ref[...], w_ref[...],
                         preferred_element_type=jnp.float32) * dinv


_prep = pl.pallas_call(
    _prep_body,
    out_shape=[
        jax.ShapeDtypeStruct((NPAD, HID), jnp.float32),
        jax.ShapeDtypeStruct((NPAD, 1), jnp.float32),
    ],
)


def _mid_body(acc_ref, p_ref, dinv_ref, b_ref, g_ref, be_ref, w_ref, out_ref):
    dinv = dinv_ref[...]
    z = dinv * (acc_ref[0] + acc_ref[1] + p_ref[...]) + b_ref[...]
    rows = lax.broadcasted_iota(jnp.int32, (NPAD, HID), 0)
    mask = rows < N
    zm = jnp.where(mask, z, 0.0)
    mu = jnp.sum(zm, axis=0, keepdims=True) * (1.0 / N)
    d = jnp.where(mask, z - mu, 0.0)
    var = jnp.sum(d * d, axis=0, keepdims=True) * (1.0 / N)
    y = g_ref[...] * (z - mu) * lax.rsqrt(var + 1e-5) + be_ref[...]
    y = jnp.maximum(y, 0.0)
    out_ref[...] = jnp.dot(y, w_ref[...],
                           preferred_element_type=jnp.float32) * dinv


_mid = pl.pallas_call(
    _mid_body,
    out_shape=jax.ShapeDtypeStruct((NPAD, HID), jnp.float32),
)


def _head_body(acc_ref, p_ref, dinv_ref, b_ref, ws_ref, bs_ref, wa_ref, ba_ref,
               s_ref, a_ref):
    z = dinv_ref[...] * (acc_ref[0] + acc_ref[1] + p_ref[...]) + b_ref[...]
    s_ref[...] = jnp.dot(z, ws_ref[...],
                         preferred_element_type=jnp.float32) + bs_ref[...]
    a_ref[...] = jnp.dot(z, wa_ref[...],
                         preferred_element_type=jnp.float32) + ba_ref[...]


_head = pl.pallas_call(
    _head_body,
    out_shape=[
        jax.ShapeDtypeStruct((NPAD, 4), jnp.float32),
        jax.ShapeDtypeStruct((NPAD, 4), jnp.float32),
    ],
)


def kernel(x, edge_index, W1, b1, W2, b2, W3, b3, gamma, beta, Ws, bs, Wa, ba):
    src2 = edge_index[0].reshape(NW, NCHUNK, CHUNK)
    dst2 = edge_index[1].reshape(NW, NCHUNK, CHUNK)
    x_pad = jnp.pad(x, ((0, NPAD - N), (0, 0)))
    ones = jnp.ones((CHUNK, DEGW), jnp.float32)
    z1 = jnp.zeros((RPT, DEGW), jnp.float32)
    z64 = jnp.zeros((RPT, HID), jnp.float32)
    b1r, b2r, b3r = b1.reshape(1, HID), b2.reshape(1, HID), b3.reshape(1, HID)
    gr, ber = gamma.reshape(1, HID), beta.reshape(1, HID)
    bsr, bar = bs.reshape(1, 4), ba.reshape(1, 4)

    deg2 = _deg_kernel(dst2, ones, z1)
    p1, dinv = _prep(x_pad, W1, deg2)

    acc1 = _edge_kernel(p1, src2, dst2, z64)
    p2 = _mid(acc1, p1, dinv, b1r, gr, ber, W2)
    acc2 = _edge_kernel(p2, src2, dst2, z64)
    p3 = _mid(acc2, p2, dinv, b2r, gr, ber, W3)
    acc3 = _edge_kernel(p3, src2, dst2, z64)
    sizes, aligns = _head(acc3, p3, dinv, b3r, Ws, bsr, Wa, bar)
    return (sizes[:N], aligns[:N])


# speedup vs baseline: 2.4691x; 1.0259x over previous
import functools

import jax
import jax.numpy as jnp
from jax import lax
from jax.experimental import pallas as pl
from jax.experimental.pallas import tpu as pltpu
from jax.experimental.pallas import tpu_sc as plsc

N = 10000
E = 320000
IN_DIM = 128
HID = 64
NC = 2
NS = 16
NW = NC * NS
CHUNK = 80
NCHUNK = 125
EPAD = NW * NCHUNK * CHUNK
NPAD = 10240
RPT = NPAD // NS

_mesh = plsc.VectorSubcoreMesh(core_axis_name="c", subcore_axis_name="s")
_sc_params = pltpu.CompilerParams(use_tc_tiling_on_sc=False)


DEGW = 16

@functools.partial(
    pl.kernel,
    out_type=jax.ShapeDtypeStruct((NC, NPAD, DEGW), jnp.float32),
    mesh=_mesh,
    scratch_types=[
        pltpu.VMEM((NCHUNK, CHUNK), jnp.int32),
        pltpu.VMEM((CHUNK, DEGW), jnp.float32),
        pltpu.VMEM_SHARED((NPAD, DEGW), jnp.float32),
        pltpu.SemaphoreType.DMA,
    ],
    compiler_params=_sc_params,
)
def _deg_kernel(dst_hbm, ones_hbm, z_hbm, out_hbm, dst_v, ones_v, deg_sp, sem):
    del sem
    cid = lax.axis_index("c")
    tid = lax.axis_index("s")
    wid = cid * NS + tid
    sl = pl.ds(tid * RPT, RPT)
    pltpu.sync_copy(z_hbm, deg_sp.at[sl])
    pltpu.sync_copy(dst_hbm.at[wid], dst_v)
    pltpu.sync_copy(ones_hbm, ones_v)
    plsc.subcore_barrier()

    def body(c, carry):
        pltpu.sync_copy(ones_v, deg_sp.at[dst_v.at[c]], add=True)
        return carry

    lax.fori_loop(0, NCHUNK, body, 0)
    plsc.subcore_barrier()
    pltpu.sync_copy(deg_sp.at[sl], out_hbm.at[cid, sl])


@functools.partial(
    pl.kernel,
    out_type=jax.ShapeDtypeStruct((NC, NPAD, HID), jnp.float32),
    mesh=_mesh,
    scratch_types=[
        pltpu.VMEM((NCHUNK, CHUNK), jnp.int32),
        pltpu.VMEM((NCHUNK, CHUNK), jnp.int32),
        pltpu.VMEM((8, CHUNK, HID), jnp.float32),
        pltpu.VMEM_SHARED((NPAD, HID), jnp.float32),
        pltpu.SemaphoreType.DMA,
        pltpu.SemaphoreType.DMA,
    ],
    compiler_params=_sc_params,
)
def _edge_kernel(p_hbm, src_hbm, dst_hbm, z_hbm, out_hbm,
                 src_v, dst_v, rows4, acc_sp, sem_g, sem_s):
    cid = lax.axis_index("c")
    tid = lax.axis_index("s")
    wid = cid * NS + tid
    sl = pl.ds(tid * RPT, RPT)
    pltpu.sync_copy(z_hbm, acc_sp.at[sl])
    pltpu.sync_copy(src_hbm.at[wid], src_v)
    pltpu.sync_copy(dst_hbm.at[wid], dst_v)
    plsc.subcore_barrier()

    AHEAD = 4

    def g_desc(c):
        return pltpu.make_async_copy(p_hbm.at[src_v.at[c]], rows4.at[c % 8],
                                     sem_g)

    def s_desc(c):
        return pltpu.make_async_copy(rows4.at[c % 8], acc_sp.at[dst_v.at[c]],
                                     sem_s)

    for i in range(AHEAD):
        g_desc(i).start()

    def body(c, carry):
        @pl.when(c >= AHEAD)
        def _():
            s_desc(c - AHEAD).wait()

        @pl.when(c + AHEAD < NCHUNK)
        def _():
            g_desc(c + AHEAD).start()

        g_desc(c).wait()
        s_desc(c).start(add=True)
        return carry

    lax.fori_loop(0, NCHUNK, body, 0)
    for i in range(AHEAD):
        s_desc(NCHUNK - AHEAD + i).wait()
    plsc.subcore_barrier()
    pltpu.sync_copy(acc_sp.at[sl], out_hbm.at[cid, sl])


def _prep_body(x_ref, w_ref, deg_ref, p_ref, dinv_ref):
    deg = deg_ref[0, :, 0:1] + deg_ref[1, :, 0:1] + 1.0
    dinv = lax.rsqrt(jnp.maximum(deg, 1.0))
    dinv_ref[...] = dinv
    p_ref[...] = jnp.dot(x_ref[...], w_ref[...],
                         preferred_element_type=jnp.float32) * dinv


_prep = pl.pallas_call(
    _prep_body,
    out_shape=[
        jax.ShapeDtypeStruct((NPAD, HID), jnp.float32),
        jax.ShapeDtypeStruct((NPAD, 1), jnp.float32),
    ],
)


def _mid_body(acc_ref, p_ref, dinv_ref, b_ref, g_ref, be_ref, w_ref, out_ref):
    dinv = dinv_ref[...]
    z = dinv * (acc_ref[0] + acc_ref[1] + p_ref[...]) + b_ref[...]
    rows = lax.broadcasted_iota(jnp.int32, (NPAD, HID), 0)
    mask = rows < N
    zm = jnp.where(mask, z, 0.0)
    mu = jnp.sum(zm, axis=0, keepdims=True) * (1.0 / N)
    d = jnp.where(mask, z - mu, 0.0)
    var = jnp.sum(d * d, axis=0, keepdims=True) * (1.0 / N)
    y = g_ref[...] * (z - mu) * lax.rsqrt(var + 1e-5) + be_ref[...]
    y = jnp.maximum(y, 0.0)
    out_ref[...] = jnp.dot(y, w_ref[...],
                           preferred_element_type=jnp.float32) * dinv


_mid = pl.pallas_call(
    _mid_body,
    out_shape=jax.ShapeDtypeStruct((NPAD, HID), jnp.float32),
)


def _head_body(acc_ref, p_ref, dinv_ref, b_ref, ws_ref, bs_ref, wa_ref, ba_ref,
               s_ref, a_ref):
    z = dinv_ref[...] * (acc_ref[0] + acc_ref[1] + p_ref[...]) + b_ref[...]
    s_ref[...] = jnp.dot(z, ws_ref[...],
                         preferred_element_type=jnp.float32) + bs_ref[...]
    a_ref[...] = jnp.dot(z, wa_ref[...],
                         preferred_element_type=jnp.float32) + ba_ref[...]


_head = pl.pallas_call(
    _head_body,
    out_shape=[
        jax.ShapeDtypeStruct((NPAD, 4), jnp.float32),
        jax.ShapeDtypeStruct((NPAD, 4), jnp.float32),
    ],
)


def kernel(x, edge_index, W1, b1, W2, b2, W3, b3, gamma, beta, Ws, bs, Wa, ba):
    src2 = edge_index[0].reshape(NW, NCHUNK, CHUNK)
    dst2 = edge_index[1].reshape(NW, NCHUNK, CHUNK)
    x_pad = jnp.pad(x, ((0, NPAD - N), (0, 0)))
    ones = jnp.ones((CHUNK, DEGW), jnp.float32)
    z1 = jnp.zeros((RPT, DEGW), jnp.float32)
    z64 = jnp.zeros((RPT, HID), jnp.float32)
    b1r, b2r, b3r = b1.reshape(1, HID), b2.reshape(1, HID), b3.reshape(1, HID)
    gr, ber = gamma.reshape(1, HID), beta.reshape(1, HID)
    bsr, bar = bs.reshape(1, 4), ba.reshape(1, 4)

    deg2 = _deg_kernel(dst2, ones, z1)
    p1, dinv = _prep(x_pad, W1, deg2)

    acc1 = _edge_kernel(p1, src2, dst2, z64)
    p2 = _mid(acc1, p1, dinv, b1r, gr, ber, W2)
    acc2 = _edge_kernel(p2, src2, dst2, z64)
    p3 = _mid(acc2, p2, dinv, b2r, gr, ber, W3)
    acc3 = _edge_kernel(p3, src2, dst2, z64)
    sizes, aligns = _head(acc3, p3, dinv, b3r, Ws, bsr, Wa, bar)
    return (sizes[:N], aligns[:N])


# speedup vs baseline: 2.5226x; 1.0217x over previous
import functools

import jax
import jax.numpy as jnp
from jax import lax
from jax.experimental import pallas as pl
from jax.experimental.pallas import tpu as pltpu
from jax.experimental.pallas import tpu_sc as plsc

N = 10000
E = 320000
IN_DIM = 128
HID = 64
NC = 2
NS = 16
NW = NC * NS
CHUNK = 80
NCHUNK = 125
EPAD = NW * NCHUNK * CHUNK
NPAD = 10240
RPT = NPAD // NS

_mesh = plsc.VectorSubcoreMesh(core_axis_name="c", subcore_axis_name="s")
_sc_params = pltpu.CompilerParams(use_tc_tiling_on_sc=False)


DEGW = 16

@functools.partial(
    pl.kernel,
    out_type=jax.ShapeDtypeStruct((NC, NPAD, DEGW), jnp.float32),
    mesh=_mesh,
    scratch_types=[
        pltpu.VMEM((NCHUNK, CHUNK), jnp.int32),
        pltpu.VMEM((CHUNK, DEGW), jnp.float32),
        pltpu.VMEM_SHARED((NPAD, DEGW), jnp.float32),
        pltpu.SemaphoreType.DMA,
    ],
    compiler_params=_sc_params,
)
def _deg_kernel(dst_hbm, ones_hbm, z_hbm, out_hbm, dst_v, ones_v, deg_sp, sem):
    cid = lax.axis_index("c")
    tid = lax.axis_index("s")
    wid = cid * NS + tid
    sl = pl.ds(tid * RPT, RPT)
    pltpu.sync_copy(z_hbm, deg_sp.at[sl])
    pltpu.sync_copy(dst_hbm.at[wid], dst_v)
    pltpu.sync_copy(ones_hbm, ones_v)
    plsc.subcore_barrier()

    def s_desc(c):
        return pltpu.make_async_copy(ones_v, deg_sp.at[dst_v.at[c]], sem)

    def body(c, carry):
        @pl.when(c >= 8)
        def _():
            s_desc(c - 8).wait()

        s_desc(c).start(add=True)
        return carry

    lax.fori_loop(0, NCHUNK, body, 0)
    for i in range(8):
        s_desc(NCHUNK - 8 + i).wait()
    plsc.subcore_barrier()
    pltpu.sync_copy(deg_sp.at[sl], out_hbm.at[cid, sl])


@functools.partial(
    pl.kernel,
    out_type=jax.ShapeDtypeStruct((NC, NPAD, HID), jnp.float32),
    mesh=_mesh,
    scratch_types=[
        pltpu.VMEM((NCHUNK, CHUNK), jnp.int32),
        pltpu.VMEM((NCHUNK, CHUNK), jnp.int32),
        pltpu.VMEM((8, CHUNK, HID), jnp.float32),
        pltpu.VMEM_SHARED((NPAD, HID), jnp.float32),
        pltpu.SemaphoreType.DMA,
        pltpu.SemaphoreType.DMA,
    ],
    compiler_params=_sc_params,
)
def _edge_kernel(p_hbm, src_hbm, dst_hbm, z_hbm, out_hbm,
                 src_v, dst_v, rows4, acc_sp, sem_g, sem_s):
    cid = lax.axis_index("c")
    tid = lax.axis_index("s")
    wid = cid * NS + tid
    sl = pl.ds(tid * RPT, RPT)
    pltpu.sync_copy(z_hbm, acc_sp.at[sl])
    pltpu.sync_copy(src_hbm.at[wid], src_v)
    pltpu.sync_copy(dst_hbm.at[wid], dst_v)
    plsc.subcore_barrier()

    AHEAD = 4

    def g_desc(c):
        return pltpu.make_async_copy(p_hbm.at[src_v.at[c]], rows4.at[c % 8],
                                     sem_g)

    def s_desc(c):
        return pltpu.make_async_copy(rows4.at[c % 8], acc_sp.at[dst_v.at[c]],
                                     sem_s)

    for i in range(AHEAD):
        g_desc(i).start()

    def body(c, carry):
        @pl.when(c >= AHEAD)
        def _():
            s_desc(c - AHEAD).wait()

        @pl.when(c + AHEAD < NCHUNK)
        def _():
            g_desc(c + AHEAD).start()

        g_desc(c).wait()
        s_desc(c).start(add=True)
        return carry

    lax.fori_loop(0, NCHUNK, body, 0)
    for i in range(AHEAD):
        s_desc(NCHUNK - AHEAD + i).wait()
    plsc.subcore_barrier()
    pltpu.sync_copy(acc_sp.at[sl], out_hbm.at[cid, sl])


def _prep_body(x_ref, w_ref, deg_ref, p_ref, dinv_ref):
    deg = deg_ref[0, :, 0:1] + deg_ref[1, :, 0:1] + 1.0
    dinv = lax.rsqrt(jnp.maximum(deg, 1.0))
    dinv_ref[...] = dinv
    p_ref[...] = jnp.dot(x_ref[...], w_ref[...],
                         preferred_element_type=jnp.float32) * dinv


_prep = pl.pallas_call(
    _prep_body,
    out_shape=[
        jax.ShapeDtypeStruct((NPAD, HID), jnp.float32),
        jax.ShapeDtypeStruct((NPAD, 1), jnp.float32),
    ],
)


def _mid_body(acc_ref, p_ref, dinv_ref, b_ref, g_ref, be_ref, w_ref, out_ref):
    dinv = dinv_ref[...]
    z = dinv * (acc_ref[0] + acc_ref[1] + p_ref[...]) + b_ref[...]
    rows = lax.broadcasted_iota(jnp.int32, (NPAD, HID), 0)
    mask = rows < N
    zm = jnp.where(mask, z, 0.0)
    mu = jnp.sum(zm, axis=0, keepdims=True) * (1.0 / N)
    d = jnp.where(mask, z - mu, 0.0)
    var = jnp.sum(d * d, axis=0, keepdims=True) * (1.0 / N)
    y = g_ref[...] * (z - mu) * lax.rsqrt(var + 1e-5) + be_ref[...]
    y = jnp.maximum(y, 0.0)
    out_ref[...] = jnp.dot(y, w_ref[...],
                           preferred_element_type=jnp.float32) * dinv


_mid = pl.pallas_call(
    _mid_body,
    out_shape=jax.ShapeDtypeStruct((NPAD, HID), jnp.float32),
)


def _head_body(acc_ref, p_ref, dinv_ref, b_ref, ws_ref, bs_ref, wa_ref, ba_ref,
               s_ref, a_ref):
    z = dinv_ref[...] * (acc_ref[0] + acc_ref[1] + p_ref[...]) + b_ref[...]
    s_ref[...] = jnp.dot(z, ws_ref[...],
                         preferred_element_type=jnp.float32) + bs_ref[...]
    a_ref[...] = jnp.dot(z, wa_ref[...],
                         preferred_element_type=jnp.float32) + ba_ref[...]


_head = pl.pallas_call(
    _head_body,
    out_shape=[
        jax.ShapeDtypeStruct((NPAD, 4), jnp.float32),
        jax.ShapeDtypeStruct((NPAD, 4), jnp.float32),
    ],
)


def kernel(x, edge_index, W1, b1, W2, b2, W3, b3, gamma, beta, Ws, bs, Wa, ba):
    src2 = edge_index[0].reshape(NW, NCHUNK, CHUNK)
    dst2 = edge_index[1].reshape(NW, NCHUNK, CHUNK)
    x_pad = jnp.pad(x, ((0, NPAD - N), (0, 0)))
    ones = jnp.ones((CHUNK, DEGW), jnp.float32)
    z1 = jnp.zeros((RPT, DEGW), jnp.float32)
    z64 = jnp.zeros((RPT, HID), jnp.float32)
    b1r, b2r, b3r = b1.reshape(1, HID), b2.reshape(1, HID), b3.reshape(1, HID)
    gr, ber = gamma.reshape(1, HID), beta.reshape(1, HID)
    bsr, bar = bs.reshape(1, 4), ba.reshape(1, 4)

    deg2 = _deg_kernel(dst2, ones, z1)
    p1, dinv = _prep(x_pad, W1, deg2)

    acc1 = _edge_kernel(p1, src2, dst2, z64)
    p2 = _mid(acc1, p1, dinv, b1r, gr, ber, W2)
    acc2 = _edge_kernel(p2, src2, dst2, z64)
    p3 = _mid(acc2, p2, dinv, b2r, gr, ber, W3)
    acc3 = _edge_kernel(p3, src2, dst2, z64)
    sizes, aligns = _head(acc3, p3, dinv, b3r, Ws, bsr, Wa, bar)
    return (sizes[:N], aligns[:N])
